# baseline (speedup 1.0000x reference)
import functools

import jax
import jax.numpy as jnp
from jax.experimental import pallas as pl
from jax.experimental.pallas import tpu as pltpu


def _conv_bd_kernel(x_ref, w_ref, b_ref, o_ref, xpad_ref, xshm_ref, xshp_ref,
                    xcol0_ref, xcol1_ref, *, H, W, K, M, C, bt, pad_rows):
    HW = H * W
    p = K // 2
    f32 = jnp.float32
    nsh = HW + 2 * W

    zeros_halo = jnp.zeros((pad_rows, C), xpad_ref.dtype)
    xpad_ref[0:pad_rows, :] = zeros_halo
    xpad_ref[pad_rows + HW:2 * pad_rows + HW, :] = zeros_halo

    row = jax.lax.broadcasted_iota(jnp.int32, (nsh, 1), 0) % W
    mask_m = (row >= 1).astype(xpad_ref.dtype)
    mask_p = (row <= W - 2).astype(xpad_ref.dtype)

    bias = b_ref[...]
    xcols = (xcol0_ref, xcol1_ref)
    for b in range(bt):
        xcol_ref = xcols[b % 2]
        xpad_ref[pad_rows:pad_rows + HW, :] = x_ref[b].astype(xpad_ref.dtype)
        xshm_ref[...] = xpad_ref[pl.ds(pad_rows - W - 1, nsh), :] * mask_m
        xshp_ref[...] = xpad_ref[pl.ds(pad_rows - W + 1, nsh), :] * mask_p
        for dy in range(K):
            dyo = dy - p
            for dx in range(K):
                t = dy * K + dx
                if dx == p:
                    win = xpad_ref[pl.ds(pad_rows + dyo * W, HW), :]
                elif dx < p:
                    win = xshm_ref[pl.ds(W + dyo * W, HW), :]
                else:
                    win = xshp_ref[pl.ds(W + dyo * W, HW), :]
                xcol_ref[:, t * C:(t + 1) * C] = win
        acc = jax.lax.dot_general(
            xcol_ref[...], w_ref[...],
            (((1,), (0,)), ((), ())),
            preferred_element_type=f32)
        o_ref[b] = (acc + bias).astype(o_ref.dtype)


def _conv_basis(x, weight, bias, basis_size, kernel_size):
    K = kernel_size
    T, N, C, H, W = x.shape
    n_basis = weight.shape[0]
    p = K // 2
    group = C // basis_size
    HW = H * W
    B = T * N
    M = group * n_basis

    pad_rows = 16 * ((p * W + p + 15) // 16)

    KKC = K * K * C
    wt = jnp.transpose(weight, (2, 3, 1, 0)).astype(jnp.bfloat16).reshape(
        K * K, 1, basis_size, n_basis)
    w_cols = jnp.tile(jnp.broadcast_to(wt, (K * K, group, basis_size,
                                            n_basis)).reshape(KKC, n_basis),
                      (1, group))
    rows = jax.lax.broadcasted_iota(jnp.int32, (KKC, M), 0)
    cols = jax.lax.broadcasted_iota(jnp.int32, (KKC, M), 1)
    keep = ((rows % C) // basis_size) == (cols // n_basis)
    w2 = jnp.where(keep, w_cols, jnp.bfloat16(0))
    b_bd = jnp.tile(bias, group).reshape(1, M).astype(jnp.float32)

    xv = jnp.transpose(x, (0, 1, 3, 4, 2)).reshape(B, HW, C)

    bt = 1
    while B % bt != 0:
        bt //= 2

    kfn = functools.partial(_conv_bd_kernel, H=H, W=W, K=K, M=M, C=C,
                            bt=bt, pad_rows=pad_rows)

    out = pl.pallas_call(
        kfn,
        out_shape=jax.ShapeDtypeStruct((B, HW, M), x.dtype),
        grid=(B // bt,),
        in_specs=[
            pl.BlockSpec((bt, HW, C), lambda i: (i, 0, 0)),
            pl.BlockSpec((K * K * C, M), lambda i: (0, 0)),
            pl.BlockSpec((1, M), lambda i: (0, 0)),
        ],
        out_specs=pl.BlockSpec((bt, HW, M), lambda i: (i, 0, 0)),
        scratch_shapes=[
            pltpu.VMEM((HW + 2 * pad_rows, C), jnp.bfloat16),
            pltpu.VMEM((HW + 2 * W, C), jnp.bfloat16),
            pltpu.VMEM((HW + 2 * W, C), jnp.bfloat16),
            pltpu.VMEM((HW, K * K * C), jnp.bfloat16),
            pltpu.VMEM((HW, K * K * C), jnp.bfloat16),
        ],
        compiler_params=pltpu.CompilerParams(
            dimension_semantics=("parallel",),
            vmem_limit_bytes=48 * 1024 * 1024,
        ),
    )(xv, w2, b_bd)

    return jnp.transpose(out.reshape(T, N, H, W, M), (0, 1, 4, 2, 3))


def kernel(x, weight, bias):
    return _conv_basis(x, weight, bias, 4, 3)


# speedup vs baseline: 1.1439x; 1.1439x over previous
import functools

import jax
import jax.numpy as jnp
from jax.experimental import pallas as pl
from jax.experimental.pallas import tpu as pltpu


def _conv_bd_kernel(x_ref, w_ref, b_ref, o_ref, xpad_ref, xshm_ref, xshp_ref,
                    xcol0_ref, xcol1_ref, *, H, W, K, M, C, bt, pad_rows):
    HW = H * W
    p = K // 2
    f32 = jnp.float32
    nsh = HW + 2 * W

    zeros_halo = jnp.zeros((pad_rows, C), xpad_ref.dtype)
    xpad_ref[0:pad_rows, :] = zeros_halo
    xpad_ref[pad_rows + HW:2 * pad_rows + HW, :] = zeros_halo

    row = jax.lax.broadcasted_iota(jnp.int32, (nsh, 1), 0) % W
    mask_m = (row >= 1).astype(xpad_ref.dtype)
    mask_p = (row <= W - 2).astype(xpad_ref.dtype)

    bias = b_ref[...]
    xcols = (xcol0_ref, xcol1_ref)
    for b in range(bt):
        xcol_ref = xcols[b % 2]
        xpad_ref[pad_rows:pad_rows + HW, :] = x_ref[b].astype(xpad_ref.dtype)
        xshm_ref[...] = xpad_ref[pl.ds(pad_rows - W - 1, nsh), :] * mask_m
        xshp_ref[...] = xpad_ref[pl.ds(pad_rows - W + 1, nsh), :] * mask_p
        for dy in range(K):
            dyo = dy - p
            for dx in range(K):
                t = dy * K + dx
                if dx == p:
                    win = xpad_ref[pl.ds(pad_rows + dyo * W, HW), :]
                elif dx < p:
                    win = xshm_ref[pl.ds(W + dyo * W, HW), :]
                else:
                    win = xshp_ref[pl.ds(W + dyo * W, HW), :]
                xcol_ref[:, t * C:(t + 1) * C] = win
        acc = jax.lax.dot_general(
            xcol_ref[...], w_ref[...],
            (((1,), (0,)), ((), ())),
            preferred_element_type=f32)
        o_ref[b] = (acc + bias).astype(o_ref.dtype)


def _conv_basis(x, weight, bias, basis_size, kernel_size):
    K = kernel_size
    T, N, C, H, W = x.shape
    n_basis = weight.shape[0]
    p = K // 2
    group = C // basis_size
    HW = H * W
    B = T * N
    M = group * n_basis

    pad_rows = 16 * ((p * W + p + 15) // 16)

    KKC = K * K * C
    wt = jnp.transpose(weight, (2, 3, 1, 0)).astype(jnp.bfloat16).reshape(
        K * K, 1, basis_size, n_basis)
    w_cols = jnp.tile(jnp.broadcast_to(wt, (K * K, group, basis_size,
                                            n_basis)).reshape(KKC, n_basis),
                      (1, group))
    rows = jax.lax.broadcasted_iota(jnp.int32, (KKC, M), 0)
    cols = jax.lax.broadcasted_iota(jnp.int32, (KKC, M), 1)
    keep = ((rows % C) // basis_size) == (cols // n_basis)
    w2 = jnp.where(keep, w_cols, jnp.bfloat16(0))
    b_bd = jnp.tile(bias, group).reshape(1, M).astype(jnp.float32)

    xv = jnp.transpose(x, (0, 1, 3, 4, 2)).reshape(B, HW, C)

    bt = 2
    while B % bt != 0:
        bt //= 2

    kfn = functools.partial(_conv_bd_kernel, H=H, W=W, K=K, M=M, C=C,
                            bt=bt, pad_rows=pad_rows)

    out = pl.pallas_call(
        kfn,
        out_shape=jax.ShapeDtypeStruct((B, HW, M), x.dtype),
        grid=(B // bt,),
        in_specs=[
            pl.BlockSpec((bt, HW, C), lambda i: (i, 0, 0)),
            pl.BlockSpec((K * K * C, M), lambda i: (0, 0)),
            pl.BlockSpec((1, M), lambda i: (0, 0)),
        ],
        out_specs=pl.BlockSpec((bt, HW, M), lambda i: (i, 0, 0)),
        scratch_shapes=[
            pltpu.VMEM((HW + 2 * pad_rows, C), jnp.bfloat16),
            pltpu.VMEM((HW + 2 * W, C), jnp.bfloat16),
            pltpu.VMEM((HW + 2 * W, C), jnp.bfloat16),
            pltpu.VMEM((HW, K * K * C), jnp.bfloat16),
            pltpu.VMEM((HW, K * K * C), jnp.bfloat16),
        ],
        compiler_params=pltpu.CompilerParams(
            dimension_semantics=("parallel",),
            vmem_limit_bytes=48 * 1024 * 1024,
        ),
    )(xv, w2, b_bd)

    return jnp.transpose(out.reshape(T, N, H, W, M), (0, 1, 4, 2, 3))


def kernel(x, weight, bias):
    return _conv_basis(x, weight, bias, 4, 3)
